# Initial kernel scaffold; baseline (speedup 1.0000x reference)
import jax
import jax.numpy as jnp
from jax.experimental import pallas as pl


def kernel(h, edge_index, Wmlp1, bmlp1, Wmlp2, bmlp2, bn_mlp_g, bn_mlp_b, bn_apply_g, bn_apply_b, bn_out_g, bn_out_b, Wpred, bpred):
    raise NotImplementedError("write your pallas kernel here")


# speedup vs baseline: 3.3483x; 3.3483x over previous
import functools

import jax
import jax.numpy as jnp
from jax import lax
from jax.experimental import pallas as pl
from jax.experimental.pallas import tpu as pltpu
from jax.experimental.pallas import tpu_sc as plsc

N = 10000
D = 128
NUM_LAYERS = 4
NC = 2
NS = 16
NW = NC * NS
NPAD = 10240
RPT = NPAD // NS
K = 128


def _make_segment_sum(ept, n_chunks):
  mesh = plsc.VectorSubcoreMesh(
      core_axis_name="c", subcore_axis_name="s", num_cores=NC, num_subcores=NS)

  @functools.partial(
      pl.kernel,
      out_type=jax.ShapeDtypeStruct((NC, NPAD, D), jnp.float32),
      mesh=mesh,
      scratch_types=[
          pltpu.VMEM((K,), jnp.int32),
          pltpu.VMEM((K,), jnp.int32),
          pltpu.VMEM((K, D), jnp.float32),
          pltpu.VMEM_SHARED((NPAD, D), jnp.float32),
          pltpu.SemaphoreType.DMA,
      ],
  )
  def seg_sum(cur_hbm, src_hbm, dst_hbm, zeros_hbm, out_hbm,
              sidx, didx, rows, acc, sem):
    c = lax.axis_index("c")
    s = lax.axis_index("s")
    tid = s * NC + c
    r0 = s * RPT
    pltpu.sync_copy(zeros_hbm.at[pl.ds(r0, RPT)], acc.at[pl.ds(r0, RPT)])
    plsc.subcore_barrier()

    def chunk(j, _):
      base = tid * ept + j * K
      pltpu.sync_copy(src_hbm.at[pl.ds(base, K)], sidx)
      pltpu.sync_copy(dst_hbm.at[pl.ds(base, K)], didx)
      pltpu.async_copy(cur_hbm.at[sidx], rows, sem).wait()
      pltpu.sync_copy(rows, acc.at[didx], add=True)
      return 0

    lax.fori_loop(0, n_chunks, chunk, 0)
    plsc.subcore_barrier()
    pltpu.sync_copy(acc.at[pl.ds(r0, RPT)], out_hbm.at[c, pl.ds(r0, RPT)])

  return seg_sum


def _tc_layer_body(cur_ref, a0_ref, a1_ref, w1_ref, b1_ref, w2_ref, b2_ref,
                   g1_ref, be1_ref, g2_ref, be2_ref, g3_ref, be3_ref,
                   out_ref, pooled_ref):
  cur = cur_ref[:]
  pooled_ref[0, :] = jnp.sum(cur, axis=0)
  x = cur + a0_ref[:] + a1_ref[:]
  y = jnp.dot(x, w1_ref[:], preferred_element_type=jnp.float32) + b1_ref[0, :]
  mu = jnp.mean(y, axis=0)
  var = jnp.mean(y * y, axis=0) - mu * mu
  y = jnp.maximum((y - mu) * lax.rsqrt(var + 1e-5) * g1_ref[0, :]
                  + be1_ref[0, :], 0.0)
  z = jnp.dot(y, w2_ref[:], preferred_element_type=jnp.float32) + b2_ref[0, :]
  mu = jnp.mean(z, axis=0)
  var = jnp.mean(z * z, axis=0) - mu * mu
  z = jnp.maximum((z - mu) * lax.rsqrt(var + 1e-5) * g2_ref[0, :]
                  + be2_ref[0, :], 0.0)
  mu = jnp.mean(z, axis=0)
  var = jnp.mean(z * z, axis=0) - mu * mu
  out_ref[:] = jnp.maximum((z - mu) * lax.rsqrt(var + 1e-5) * g3_ref[0, :]
                           + be3_ref[0, :], 0.0)


_tc_layer = pl.pallas_call(
    _tc_layer_body,
    out_shape=(
        jax.ShapeDtypeStruct((N, D), jnp.float32),
        jax.ShapeDtypeStruct((1, D), jnp.float32),
    ),
)


def _pred_body(h_ref, pooled_ref, wp_ref, bp_ref, out_ref):
  p_last = jnp.sum(h_ref[:], axis=0).reshape(1, D)
  score = jnp.zeros((1, wp_ref.shape[2]), jnp.float32)
  for i in range(NUM_LAYERS):
    score = score + jnp.dot(pooled_ref[i].reshape(1, D), wp_ref[i],
                            preferred_element_type=jnp.float32) + bp_ref[i]
  score = score + jnp.dot(p_last, wp_ref[NUM_LAYERS],
                          preferred_element_type=jnp.float32) \
      + bp_ref[NUM_LAYERS]
  out_ref[:] = score


def kernel(h, edge_index, Wmlp1, bmlp1, Wmlp2, bmlp2, bn_mlp_g, bn_mlp_b,
           bn_apply_g, bn_apply_b, bn_out_g, bn_out_b, Wpred, bpred):
  E = edge_index.shape[1]
  ept = -(-E // (NW * K)) * K
  e_pad = ept * NW
  src = edge_index[0]
  dst = edge_index[1]
  if e_pad > E:
    src = jnp.concatenate([src, jnp.zeros((e_pad - E,), jnp.int32)])
    dst = jnp.concatenate([dst, jnp.full((e_pad - E,), N, jnp.int32)])
  zeros = jnp.zeros((NPAD, D), jnp.float32)
  seg_sum = _make_segment_sum(ept, ept // K)

  def row(v, i):
    return v[i].reshape(1, D)

  pooled = []
  cur = h
  for i in range(NUM_LAYERS):
    acc = seg_sum(cur, src, dst, zeros)
    cur, p = _tc_layer(cur, acc[0, :N], acc[1, :N], Wmlp1[i], row(bmlp1, i),
                       Wmlp2[i], row(bmlp2, i), row(bn_mlp_g, i),
                       row(bn_mlp_b, i), row(bn_apply_g, i),
                       row(bn_apply_b, i), row(bn_out_g, i), row(bn_out_b, i))
    pooled.append(p)
  pooled = jnp.concatenate(pooled, axis=0)

  pred = pl.pallas_call(
      _pred_body,
      out_shape=jax.ShapeDtypeStruct((1, Wpred.shape[2]), jnp.float32),
  )
  return pred(cur, pooled, Wpred, bpred)
